# baseline (speedup 1.0000x reference)
import functools

import jax
import jax.numpy as jnp
from jax import lax
from jax.experimental import pallas as pl
from jax.experimental.pallas import tpu as pltpu
from jax.experimental.pallas import tpu_sc as plsc

NUM_CLASSES = 1000
HIDDEN = 128
BATCH = 16384

_B_SC = 8192
_B_TC = BATCH - _B_SC

_info = plsc.get_sparse_core_info()
_NC, _NS = _info.num_cores, _info.num_subcores
_NW = _NC * _NS
_BPW = _B_SC // _NW
_CHUNK = 128
_NCHUNK = _BPW // _CHUNK

_mesh = plsc.VectorSubcoreMesh(core_axis_name="c", subcore_axis_name="s")


@functools.partial(
    pl.kernel,
    mesh=_mesh,
    out_type=jax.ShapeDtypeStruct((_B_SC, HIDDEN), jnp.float32),
    scratch_types=[
        pltpu.VMEM((_NCHUNK, _CHUNK), jnp.int32),
        pltpu.VMEM((_BPW, HIDDEN), jnp.float32),
        [pltpu.SemaphoreType.DMA for _ in range(_NCHUNK)],
        pltpu.SemaphoreType.DMA,
    ],
)
def _sc_embed(table_hbm, labels_hbm, out_hbm, idx_v, rows_v, gsems, wsem):
    wid = lax.axis_index("s") * _NC + lax.axis_index("c")
    pltpu.sync_copy(labels_hbm.at[wid], idx_v)
    gathers = []
    for j in range(_NCHUNK):
        gathers.append(
            pltpu.async_copy(
                table_hbm.at[idx_v.at[j]],
                rows_v.at[pl.ds(j * _CHUNK, _CHUNK)],
                gsems[j],
            )
        )
    writes = []
    for j in range(_NCHUNK):
        gathers[j].wait()
        writes.append(
            pltpu.async_copy(
                rows_v.at[pl.ds(j * _CHUNK, _CHUNK)],
                out_hbm.at[pl.ds(wid * _BPW + j * _CHUNK, _CHUNK)],
                wsem,
            )
        )
    for w in writes:
        w.wait()


_VPAD = 1024
_BB = 1024


def _tc_body(labels_ref, table_ref, out_ref):
    labels = labels_ref[0, 0, :]
    onehot = (labels[:, None] == lax.broadcasted_iota(
        jnp.int32, (_BB, _VPAD), 1)).astype(jnp.float32)
    out_ref[...] = jnp.dot(onehot, table_ref[...],
                           preferred_element_type=jnp.float32)


def _tc_embed(labels_tc, table_pad):
    grid = (_B_TC // _BB,)
    return pl.pallas_call(
        _tc_body,
        grid=grid,
        in_specs=[
            pl.BlockSpec((1, 1, _BB), lambda i: (i, 0, 0)),
            pl.BlockSpec((_VPAD, HIDDEN), lambda i: (0, 0)),
        ],
        out_specs=pl.BlockSpec((_BB, HIDDEN), lambda i: (i, 0)),
        out_shape=jax.ShapeDtypeStruct((_B_TC, HIDDEN), jnp.float32),
    )(labels_tc.reshape(_B_TC // _BB, 1, _BB), table_pad)


def kernel(labels, train, dtype, table):
    labels = labels.astype(jnp.int32)
    labels_sc = labels[:_B_SC].reshape(_NW, _NCHUNK, _CHUNK)
    table_pad = jnp.pad(table, ((0, _VPAD - table.shape[0]), (0, 0)))
    out_sc = _sc_embed(table, labels_sc)
    out_tc = _tc_embed(labels[_B_SC:], table_pad)
    out = jnp.concatenate([out_sc, out_tc], axis=0)
    return out.astype(dtype.dtype)


# speedup vs baseline: 1.1119x; 1.1119x over previous
import functools

import jax
import jax.numpy as jnp
from jax import lax
from jax.experimental import pallas as pl
from jax.experimental.pallas import tpu as pltpu
from jax.experimental.pallas import tpu_sc as plsc

NUM_CLASSES = 1000
HIDDEN = 128
BATCH = 16384

_B_SC = 8192
_B_TC = BATCH - _B_SC

_info = plsc.get_sparse_core_info()
_NC, _NS = _info.num_cores, _info.num_subcores
_NW = _NC * _NS
_BPW = _B_SC // _NW
_CHUNK = 128
_NCHUNK = _BPW // _CHUNK
_NROW = BATCH // (_NCHUNK * _CHUNK)

_mesh = plsc.VectorSubcoreMesh(core_axis_name="c", subcore_axis_name="s")


@functools.partial(
    pl.kernel,
    mesh=_mesh,
    out_type=jax.ShapeDtypeStruct((_B_SC, HIDDEN), jnp.float32),
    scratch_types=[
        pltpu.VMEM((_NCHUNK, _CHUNK), jnp.int32),
        pltpu.VMEM((_BPW, HIDDEN), jnp.float32),
        [pltpu.SemaphoreType.DMA for _ in range(_NCHUNK)],
        pltpu.SemaphoreType.DMA,
    ],
)
def _sc_embed(table_hbm, labels_hbm, out_hbm, idx_v, rows_v, gsems, wsem):
    wid = lax.axis_index("s") * _NC + lax.axis_index("c")
    pltpu.sync_copy(labels_hbm.at[wid], idx_v)
    gathers = []
    for j in range(_NCHUNK):
        gathers.append(
            pltpu.async_copy(
                table_hbm.at[idx_v.at[j]],
                rows_v.at[pl.ds(j * _CHUNK, _CHUNK)],
                gsems[j],
            )
        )
    writes = []
    for j in range(_NCHUNK):
        gathers[j].wait()
        writes.append(
            pltpu.async_copy(
                rows_v.at[pl.ds(j * _CHUNK, _CHUNK)],
                out_hbm.at[pl.ds(wid * _BPW + j * _CHUNK, _CHUNK)],
                wsem,
            )
        )
    for w in writes:
        w.wait()


_VPAD = 1024
_BB = 1024
_TC_BLK0 = _B_SC // _BB


def _tc_body(labels_ref, table_ref, out_ref):
    labels = labels_ref[0, 0, :]
    onehot = (labels[:, None] == lax.broadcasted_iota(
        jnp.int32, (_BB, _VPAD), 1)).astype(jnp.bfloat16)
    out_ref[...] = jnp.dot(onehot, table_ref[...],
                           preferred_element_type=jnp.float32)


def _tc_embed(labels_full, table_bf):
    grid = (_B_TC // _BB,)
    return pl.pallas_call(
        _tc_body,
        grid=grid,
        in_specs=[
            pl.BlockSpec((1, 1, _BB), lambda i: (i + _TC_BLK0, 0, 0)),
            pl.BlockSpec((_VPAD, HIDDEN), lambda i: (0, 0)),
        ],
        out_specs=pl.BlockSpec((_BB, HIDDEN), lambda i: (i + _TC_BLK0, 0)),
        out_shape=jax.ShapeDtypeStruct((BATCH, HIDDEN), jnp.float32),
    )(labels_full.reshape(BATCH // _BB, 1, _BB), table_bf)


def kernel(labels, train, dtype, table):
    labels = labels.astype(jnp.int32)
    labels_sc = labels.reshape(_NROW, _NCHUNK, _CHUNK)
    table_bf = jnp.pad(table, ((0, _VPAD - table.shape[0]), (0, 0))).astype(
        jnp.bfloat16)
    out_sc = _sc_embed(table, labels_sc)
    out_full = _tc_embed(labels, table_bf)
    out = lax.dynamic_update_slice(out_full, out_sc, (0, 0))
    return out.astype(dtype.dtype)


# speedup vs baseline: 1.1142x; 1.0020x over previous
import functools

import jax
import jax.numpy as jnp
from jax import lax
from jax.experimental import pallas as pl
from jax.experimental.pallas import tpu as pltpu
from jax.experimental.pallas import tpu_sc as plsc

NUM_CLASSES = 1000
HIDDEN = 128
BATCH = 16384

_B_SC = 8192
_B_TC = BATCH - _B_SC

_info = plsc.get_sparse_core_info()
_NC, _NS = _info.num_cores, _info.num_subcores
_NW = _NC * _NS
_BPW = _B_SC // _NW
_CHUNK = 128
_NCHUNK = _BPW // _CHUNK
_NROW = BATCH // (_NCHUNK * _CHUNK)

_mesh = plsc.VectorSubcoreMesh(core_axis_name="c", subcore_axis_name="s")


@functools.partial(
    pl.kernel,
    mesh=_mesh,
    out_type=jax.ShapeDtypeStruct((BATCH, HIDDEN), jnp.float32),
    scratch_types=[
        pltpu.VMEM((_NCHUNK, _CHUNK), jnp.int32),
        pltpu.VMEM((_BPW, HIDDEN), jnp.float32),
        [pltpu.SemaphoreType.DMA for _ in range(_NCHUNK)],
        pltpu.SemaphoreType.DMA,
    ],
)
def _sc_embed(table_hbm, labels_hbm, out_hbm, idx_v, rows_v, gsems, wsem):
    wid = lax.axis_index("s") * _NC + lax.axis_index("c")
    pltpu.sync_copy(labels_hbm.at[wid], idx_v)
    gathers = []
    for j in range(_NCHUNK):
        gathers.append(
            pltpu.async_copy(
                table_hbm.at[idx_v.at[j]],
                rows_v.at[pl.ds(j * _CHUNK, _CHUNK)],
                gsems[j],
            )
        )
    writes = []
    for j in range(_NCHUNK):
        gathers[j].wait()
        writes.append(
            pltpu.async_copy(
                rows_v.at[pl.ds(j * _CHUNK, _CHUNK)],
                out_hbm.at[pl.ds(wid * _BPW + j * _CHUNK, _CHUNK)],
                wsem,
            )
        )
    for w in writes:
        w.wait()


_VPAD = 1024
_BB = 1024
_TC_BLK0 = _B_SC // _BB


def _tc_body(labels_ref, table_ref, out_ref):
    labels = labels_ref[0, 0, :]
    onehot = (labels[:, None] == lax.broadcasted_iota(
        jnp.int32, (_BB, _VPAD), 1)).astype(jnp.bfloat16)
    out_ref[...] = jnp.dot(onehot, table_ref[...],
                           preferred_element_type=jnp.float32)


def _tc_embed(labels_full, table_bf):
    grid = (_B_TC // _BB,)
    return pl.pallas_call(
        _tc_body,
        grid=grid,
        in_specs=[
            pl.BlockSpec((1, 1, _BB), lambda i: (i + _TC_BLK0, 0, 0)),
            pl.BlockSpec((_VPAD, HIDDEN), lambda i: (0, 0)),
        ],
        out_specs=pl.BlockSpec((_BB, HIDDEN), lambda i: (i, 0)),
        out_shape=jax.ShapeDtypeStruct((_B_TC, HIDDEN), jnp.float32),
    )(labels_full.reshape(BATCH // _BB, 1, _BB), table_bf)


def kernel(labels, train, dtype, table):
    labels = labels.astype(jnp.int32)
    labels_sc = labels.reshape(_NROW, _NCHUNK, _CHUNK)
    table_bf = jnp.pad(table, ((0, _VPAD - table.shape[0]), (0, 0))).astype(
        jnp.bfloat16)
    out_sc_full = _sc_embed(table, labels_sc)
    out_tc = _tc_embed(labels, table_bf)
    out = lax.dynamic_update_slice(out_sc_full, out_tc, (_B_SC, 0))
    return out.astype(dtype.dtype)


# speedup vs baseline: 1.1656x; 1.0461x over previous
import functools

import jax
import jax.numpy as jnp
from jax import lax
from jax.experimental import pallas as pl
from jax.experimental.pallas import tpu as pltpu
from jax.experimental.pallas import tpu_sc as plsc

NUM_CLASSES = 1000
HIDDEN = 128
BATCH = 16384

_info = plsc.get_sparse_core_info()
_NC, _NS = _info.num_cores, _info.num_subcores
_NW = _NC * _NS
_BPW = BATCH // _NW
_CHUNK = 128
_NCHUNK = _BPW // _CHUNK
_HALF = _NCHUNK // 2

_mesh = plsc.VectorSubcoreMesh(core_axis_name="c", subcore_axis_name="s")


@functools.partial(
    pl.kernel,
    mesh=_mesh,
    out_type=jax.ShapeDtypeStruct((BATCH, HIDDEN), jnp.float32),
    scratch_types=[
        pltpu.VMEM((_NCHUNK, _CHUNK), jnp.int32),
        pltpu.VMEM((_BPW, HIDDEN), jnp.float32),
        [pltpu.SemaphoreType.DMA for _ in range(_NCHUNK)],
        pltpu.SemaphoreType.DMA,
    ],
)
def _sc_embed(table_hbm, labels_hbm, out_hbm, idx_v, rows_v, gsems, wsem):
    wid = lax.axis_index("c") * _NS + lax.axis_index("s")
    pltpu.sync_copy(labels_hbm.at[wid], idx_v)
    gathers = []
    for j in range(_NCHUNK):
        gathers.append(
            pltpu.async_copy(
                table_hbm.at[idx_v.at[j]],
                rows_v.at[pl.ds(j * _CHUNK, _CHUNK)],
                gsems[j],
            )
        )
    writes = []
    for h in range(2):
        for j in range(h * _HALF, (h + 1) * _HALF):
            gathers[j].wait()
        writes.append(
            pltpu.async_copy(
                rows_v.at[pl.ds(h * _HALF * _CHUNK, _HALF * _CHUNK)],
                out_hbm.at[pl.ds(wid * _BPW + h * _HALF * _CHUNK,
                                 _HALF * _CHUNK)],
                wsem,
            )
        )
    for w in writes:
        w.wait()


def kernel(labels, train, dtype, table):
    labels3d = labels.astype(jnp.int32).reshape(_NW, _NCHUNK, _CHUNK)
    out = _sc_embed(table, labels3d)
    return out.astype(dtype.dtype)


# speedup vs baseline: 1.3023x; 1.1173x over previous
import functools

import jax
import jax.numpy as jnp
from jax import lax
from jax.experimental import pallas as pl
from jax.experimental.pallas import tpu as pltpu
from jax.experimental.pallas import tpu_sc as plsc

NUM_CLASSES = 1000
HIDDEN = 128
BATCH = 16384

_info = plsc.get_sparse_core_info()
_NC, _NS = _info.num_cores, _info.num_subcores
_NW = _NC * _NS
_BPW = BATCH // _NW
_CHUNK = 128
_NCHUNK = _BPW // _CHUNK

_mesh = plsc.VectorSubcoreMesh(core_axis_name="c", subcore_axis_name="s")


@functools.partial(
    pl.kernel,
    mesh=_mesh,
    out_type=jax.ShapeDtypeStruct((BATCH, HIDDEN), jnp.float32),
    scratch_types=[
        pltpu.VMEM((_NCHUNK, _CHUNK), jnp.int32),
        pltpu.VMEM((_BPW, HIDDEN), jnp.float32),
        pltpu.VMEM_SHARED((NUM_CLASSES + 1, HIDDEN), jnp.float32),
        [pltpu.SemaphoreType.DMA for _ in range(_NCHUNK)],
        pltpu.SemaphoreType.DMA,
    ],
)
def _sc_embed(table_hbm, labels_hbm, out_hbm, idx_v, rows_v, tab_sh,
              gsems, wsem):
    sid = lax.axis_index("s")
    wid = lax.axis_index("c") * _NS + sid
    start = pl.multiple_of(sid * 64, 64)

    @pl.when(sid < 15)
    def _():
        pltpu.sync_copy(table_hbm.at[pl.ds(start, 64)],
                        tab_sh.at[pl.ds(start, 64)])

    @pl.when(sid == 15)
    def _():
        pltpu.sync_copy(table_hbm.at[pl.ds(960, NUM_CLASSES + 1 - 960)],
                        tab_sh.at[pl.ds(960, NUM_CLASSES + 1 - 960)])

    pltpu.sync_copy(labels_hbm.at[wid], idx_v)
    plsc.subcore_barrier()
    gathers = []
    for j in range(_NCHUNK):
        gathers.append(
            pltpu.async_copy(
                tab_sh.at[idx_v.at[j]],
                rows_v.at[pl.ds(j * _CHUNK, _CHUNK)],
                gsems[j],
            )
        )
    for g in gathers:
        g.wait()
    pltpu.sync_copy(rows_v, out_hbm.at[pl.ds(wid * _BPW, _BPW)])


def kernel(labels, train, dtype, table):
    labels3d = labels.astype(jnp.int32).reshape(_NW, _NCHUNK, _CHUNK)
    out = _sc_embed(table, labels3d)
    return out.astype(dtype.dtype)


# speedup vs baseline: 1.3606x; 1.0448x over previous
import functools

import jax
import jax.numpy as jnp
from jax import lax
from jax.experimental import pallas as pl
from jax.experimental.pallas import tpu as pltpu
from jax.experimental.pallas import tpu_sc as plsc

NUM_CLASSES = 1000
HIDDEN = 128
BATCH = 16384

_info = plsc.get_sparse_core_info()
_NC, _NS = _info.num_cores, _info.num_subcores
_NW = _NC * _NS
_BPW = BATCH // _NW
_CHUNK = 128
_NCHUNK = _BPW // _CHUNK

_mesh = plsc.VectorSubcoreMesh(core_axis_name="c", subcore_axis_name="s")


@functools.partial(
    pl.kernel,
    mesh=_mesh,
    out_type=jax.ShapeDtypeStruct((BATCH, HIDDEN), jnp.float32),
    scratch_types=[
        pltpu.VMEM((_NCHUNK, _CHUNK), jnp.int32),
        pltpu.VMEM((_BPW, HIDDEN), jnp.float32),
        pltpu.VMEM_SHARED((NUM_CLASSES + 1, HIDDEN), jnp.float32),
        [pltpu.SemaphoreType.DMA for _ in range(_NCHUNK)],
        pltpu.SemaphoreType.DMA,
    ],
)
def _sc_embed(table_hbm, labels_hbm, out_hbm, idx_v, rows_v, tab_sh,
              gsems, wsem):
    sid = lax.axis_index("s")
    wid = lax.axis_index("c") * _NS + sid
    pltpu.sync_copy(labels_hbm.at[wid], idx_v)
    start = pl.multiple_of(sid * 64, 64)

    @pl.when(sid < 15)
    def _():
        pltpu.sync_copy(table_hbm.at[pl.ds(start, 64)],
                        tab_sh.at[pl.ds(start, 64)])

    @pl.when(sid == 15)
    def _():
        pltpu.sync_copy(table_hbm.at[pl.ds(960, NUM_CLASSES + 1 - 960)],
                        tab_sh.at[pl.ds(960, NUM_CLASSES + 1 - 960)])

    plsc.subcore_barrier()
    gathers = []
    for j in range(_NCHUNK):
        gathers.append(
            pltpu.async_copy(
                tab_sh.at[idx_v.at[j]],
                rows_v.at[pl.ds(j * _CHUNK, _CHUNK)],
                gsems[j],
            )
        )
    writes = []
    for j in range(_NCHUNK):
        gathers[j].wait()
        writes.append(
            pltpu.async_copy(
                rows_v.at[pl.ds(j * _CHUNK, _CHUNK)],
                out_hbm.at[pl.ds(wid * _BPW + j * _CHUNK, _CHUNK)],
                wsem,
            )
        )
    for w in writes:
        w.wait()


def kernel(labels, train, dtype, table):
    labels3d = labels.astype(jnp.int32).reshape(_NW, _NCHUNK, _CHUNK)
    out = _sc_embed(table, labels3d)
    return out.astype(dtype.dtype)


# speedup vs baseline: 1.3709x; 1.0076x over previous
import functools

import jax
import jax.numpy as jnp
from jax import lax
from jax.experimental import pallas as pl
from jax.experimental.pallas import tpu as pltpu
from jax.experimental.pallas import tpu_sc as plsc

NUM_CLASSES = 1000
HIDDEN = 128
BATCH = 16384

_info = plsc.get_sparse_core_info()
_NC, _NS = _info.num_cores, _info.num_subcores
_NW = _NC * _NS
_BPW = BATCH // _NW
_CHUNK = 64
_NCHUNK = _BPW // _CHUNK
_NHBM = 2

_mesh = plsc.VectorSubcoreMesh(core_axis_name="c", subcore_axis_name="s")


@functools.partial(
    pl.kernel,
    mesh=_mesh,
    out_type=jax.ShapeDtypeStruct((BATCH, HIDDEN), jnp.float32),
    scratch_types=[
        pltpu.VMEM((_NCHUNK, _CHUNK), jnp.int32),
        pltpu.VMEM((_BPW, HIDDEN), jnp.float32),
        pltpu.VMEM_SHARED((NUM_CLASSES + 1, HIDDEN), jnp.float32),
        [pltpu.SemaphoreType.DMA for _ in range(_NCHUNK)],
        pltpu.SemaphoreType.DMA,
        pltpu.SemaphoreType.DMA,
    ],
)
def _sc_embed(table_hbm, labels_hbm, out_hbm, idx_v, rows_v, tab_sh,
              gsems, wsem, ssem):
    sid = lax.axis_index("s")
    wid = lax.axis_index("c") * _NS + sid
    start = pl.multiple_of(sid * 64, 8)
    stage64 = pltpu.make_async_copy(
        table_hbm.at[pl.ds(start, 64)], tab_sh.at[pl.ds(start, 64)], ssem)
    stage41 = pltpu.make_async_copy(
        table_hbm.at[pl.ds(960, NUM_CLASSES + 1 - 960)],
        tab_sh.at[pl.ds(960, NUM_CLASSES + 1 - 960)], ssem)

    @pl.when(sid < 15)
    def _():
        stage64.start()

    @pl.when(sid == 15)
    def _():
        stage41.start()

    pltpu.sync_copy(labels_hbm.at[wid], idx_v)

    gathers = []
    for j in range(_NHBM):
        gathers.append(
            pltpu.async_copy(
                table_hbm.at[idx_v.at[j]],
                rows_v.at[pl.ds(j * _CHUNK, _CHUNK)],
                gsems[j],
            )
        )

    @pl.when(sid < 15)
    def _():
        stage64.wait()

    @pl.when(sid == 15)
    def _():
        stage41.wait()

    plsc.subcore_barrier()
    for j in range(_NHBM, _NCHUNK):
        gathers.append(
            pltpu.async_copy(
                tab_sh.at[idx_v.at[j]],
                rows_v.at[pl.ds(j * _CHUNK, _CHUNK)],
                gsems[j],
            )
        )
    writes = []
    for j in range(_NCHUNK):
        gathers[j].wait()
        writes.append(
            pltpu.async_copy(
                rows_v.at[pl.ds(j * _CHUNK, _CHUNK)],
                out_hbm.at[pl.ds(wid * _BPW + j * _CHUNK, _CHUNK)],
                wsem,
            )
        )
    for w in writes:
        w.wait()


def kernel(labels, train, dtype, table):
    labels3d = labels.astype(jnp.int32).reshape(_NW, _NCHUNK, _CHUNK)
    out = _sc_embed(table, labels3d)
    return out.astype(dtype.dtype)
